# baseline (speedup 1.0000x reference)
import functools

import jax
import jax.numpy as jnp
from jax import lax
from jax.experimental import pallas as pl
from jax.experimental.pallas import tpu as pltpu
from jax.experimental.pallas import tpu_sc as plsc

N_COLS = 256
N_ROWS = 65536
N_SEL = 64

NUM_CORES = 2
NUM_SUBCORES = 16
NW = NUM_CORES * NUM_SUBCORES
CHUNK = N_ROWS // NW
BATCH = 8
NBATCH = N_SEL // BATCH
NBUF = 4


def _sc_gather(values2, cols):
  mesh = plsc.VectorSubcoreMesh(core_axis_name="c", subcore_axis_name="s")

  @functools.partial(
      pl.kernel,
      mesh=mesh,
      out_type=jax.ShapeDtypeStruct((N_SEL, N_ROWS), jnp.float32),
      scratch_types=[
          pltpu.VMEM((N_SEL,), jnp.int32),
          pltpu.VMEM((NBUF, BATCH, CHUNK), jnp.float32),
          pltpu.SemaphoreType.DMA,
          pltpu.SemaphoreType.DMA,
      ],
  )
  def k(values_hbm, cols_hbm, out_hbm, idx_v, buf_v, gsem, ssem):
    w = lax.axis_index("s") * NUM_CORES + lax.axis_index("c")
    col0 = w * CHUNK

    pltpu.sync_copy(cols_hbm, idx_v)
    for i in range(N_SEL // 16):
      sl = pl.ds(i * 16, 16)
      idx_v[sl] = idx_v[sl] * NW + w

    def gather(b):
      c = pltpu.make_async_copy(
          values_hbm.at[idx_v.at[pl.ds(b * BATCH, BATCH)]],
          buf_v.at[b % NBUF],
          gsem,
      )
      c.start()
      return c

    def scatter(b):
      s = pltpu.make_async_copy(
          buf_v.at[b % NBUF],
          out_hbm.at[pl.ds(b * BATCH, BATCH), pl.ds(col0, CHUNK)],
          ssem,
      )
      s.start()
      return s

    gathers = [gather(b) for b in range(NBUF)]
    scatters = []
    for b in range(NBATCH):
      gathers[b].wait()
      scatters.append(scatter(b))
      if b + NBUF < NBATCH:
        scatters[b].wait()
        gathers.append(gather(b + NBUF))
    for b in range(NBATCH - NBUF, NBATCH):
      scatters[b].wait()

  return k(values2, cols)


def kernel(values, cols):
  values2 = values.reshape(N_COLS * NW, CHUNK)
  return _sc_gather(values2, cols)


# speedup vs baseline: 2.7186x; 2.7186x over previous
import functools

import jax
import jax.numpy as jnp
from jax import lax
from jax.experimental import pallas as pl
from jax.experimental.pallas import tpu as pltpu
from jax.experimental.pallas import tpu_sc as plsc

N_COLS = 256
N_ROWS = 65536
N_SEL = 64

NUM_CORES = 2
NUM_SUBCORES = 16
NW = NUM_CORES * NUM_SUBCORES
CHUNK = N_ROWS // NW
BATCH = 8
NBATCH = N_SEL // BATCH
NBUF = 4


def _sc_gather(values2, cols):
  mesh = plsc.VectorSubcoreMesh(core_axis_name="c", subcore_axis_name="s")

  @functools.partial(
      pl.kernel,
      mesh=mesh,
      out_type=jax.ShapeDtypeStruct((N_SEL, N_ROWS), jnp.float32),
      scratch_types=[
          pltpu.VMEM((N_SEL,), jnp.int32),
          pltpu.VMEM((NBUF, BATCH, CHUNK), jnp.float32),
          pltpu.SemaphoreType.DMA,
          pltpu.SemaphoreType.DMA,
      ],
  )
  def k(values_hbm, cols_hbm, out_hbm, idx_v, buf_v, gsem, ssem):
    w = lax.axis_index("s") * NUM_CORES + lax.axis_index("c")
    col0 = w * CHUNK

    pltpu.sync_copy(cols_hbm, idx_v)

    def gather(b):
      c = pltpu.make_async_copy(
          values_hbm.at[idx_v.at[pl.ds(b * BATCH, BATCH)], pl.ds(col0, CHUNK)],
          buf_v.at[b % NBUF],
          gsem,
      )
      c.start()
      return c

    def scatter(b):
      s = pltpu.make_async_copy(
          buf_v.at[b % NBUF],
          out_hbm.at[pl.ds(b * BATCH, BATCH), pl.ds(col0, CHUNK)],
          ssem,
      )
      s.start()
      return s

    gathers = [gather(b) for b in range(NBUF)]
    scatters = []
    for b in range(NBATCH):
      gathers[b].wait()
      scatters.append(scatter(b))
      if b + NBUF < NBATCH:
        scatters[b].wait()
        gathers.append(gather(b + NBUF))
    for b in range(NBATCH - NBUF, NBATCH):
      scatters[b].wait()

  return k(values2, cols)


def kernel(values, cols):
  return _sc_gather(values, cols)


# speedup vs baseline: 2.8080x; 1.0329x over previous
import functools

import jax
import jax.numpy as jnp
from jax import lax
from jax.experimental import pallas as pl
from jax.experimental.pallas import tpu as pltpu
from jax.experimental.pallas import tpu_sc as plsc

N_COLS = 256
N_ROWS = 65536
N_SEL = 64

NUM_CORES = 2
NUM_SUBCORES = 16
NW = NUM_CORES * NUM_SUBCORES
CHUNK = N_ROWS // NW
BATCH = 8
NBATCH = N_SEL // BATCH
NBUF = 7


def _sc_gather(values2, cols):
  mesh = plsc.VectorSubcoreMesh(core_axis_name="c", subcore_axis_name="s")

  @functools.partial(
      pl.kernel,
      mesh=mesh,
      out_type=jax.ShapeDtypeStruct((N_SEL, N_ROWS), jnp.float32),
      scratch_types=[
          pltpu.VMEM((N_SEL,), jnp.int32),
          pltpu.VMEM((NBUF, BATCH, CHUNK), jnp.float32),
          pltpu.SemaphoreType.DMA,
          pltpu.SemaphoreType.DMA,
      ],
  )
  def k(values_hbm, cols_hbm, out_hbm, idx_v, buf_v, gsem, ssem):
    w = lax.axis_index("s") * NUM_CORES + lax.axis_index("c")
    col0 = w * CHUNK

    pltpu.sync_copy(cols_hbm, idx_v)

    def gather(b):
      c = pltpu.make_async_copy(
          values_hbm.at[idx_v.at[pl.ds(b * BATCH, BATCH)], pl.ds(col0, CHUNK)],
          buf_v.at[b % NBUF],
          gsem,
      )
      c.start()
      return c

    def scatter(b):
      s = pltpu.make_async_copy(
          buf_v.at[b % NBUF],
          out_hbm.at[pl.ds(b * BATCH, BATCH), pl.ds(col0, CHUNK)],
          ssem,
      )
      s.start()
      return s

    gathers = [gather(b) for b in range(NBUF)]
    scatters = []
    for b in range(NBATCH):
      gathers[b].wait()
      scatters.append(scatter(b))
      if b + NBUF < NBATCH:
        scatters[b].wait()
        gathers.append(gather(b + NBUF))
    for b in range(NBATCH - NBUF, NBATCH):
      scatters[b].wait()

  return k(values2, cols)


def kernel(values, cols):
  return _sc_gather(values, cols)
